# baseline (speedup 1.0000x reference)
import jax
import jax.numpy as jnp
from jax import lax
from jax.experimental import pallas as pl
from jax.experimental.pallas import tpu as pltpu
from jax.experimental.pallas import tpu_sc as plsc

IN_C = 4096
OUT_C = 4096
INW = IN_C // 2
TOTW = OUT_C * INW
NC = 2
NS = 16
NW = NC * NS
REG = TOTW // NW
CHUNK = 128
BLK = 64
PSZ = 32768
PROWS = PSZ // INW
NPIECE = REG // PSZ


def _sc_densify_body(idx_hbm, val_hbm, starts_hbm, wout_hbm,
                     idx_buf, val_buf, pbuf0, pbuf1, starts_v, sem0, sem1):
    wid = lax.axis_index("s") * NC + lax.axis_index("c")
    sems = (sem0, sem1)
    pbufs = (pbuf0, pbuf1)

    pltpu.sync_copy(starts_hbm, starts_v)
    srow = starts_v[wid, :]
    ps = [srow[i] for i in range(NPIECE + 1)]
    c_lo = (ps[0] // CHUNK) // 8 * 8
    c_hi = (ps[NPIECE] + CHUNK - 1) // CHUNK
    nblk = (c_hi - c_lo + BLK - 1) // BLK

    def _stage(blk):
        pltpu.sync_copy(idx_hbm.at[pl.ds(blk, BLK)], idx_buf)
        pltpu.sync_copy(val_hbm.at[pl.ds(blk, BLK)], val_buf)

    _stage(c_lo)

    for p in range(NPIECE):
        b = p % 2
        pb = pbufs[b]
        plo = wid * REG + p * PSZ
        prow = wid * (NPIECE * PROWS) + p * PROWS

        if p >= 2:
            prev = wid * (NPIECE * PROWS) + (p - 2) * PROWS
            pltpu.make_async_copy(
                pb.at[pl.ds(0, PROWS)], wout_hbm.at[pl.ds(prev, PROWS)],
                sems[b]).wait()
        if p > 0:
            @pl.when(nblk > 1)
            def _():
                _stage(c_lo)

        def _zf(i, c):
            for r in range(PROWS):
                pb[r, pl.ds(i * 16, 16)] = jnp.zeros((16,), jnp.int32)
            return c

        lax.fori_loop(0, INW // 16, _zf, 0)

        lo_v = jnp.full((16,), plo, jnp.int32)
        hi_v = jnp.full((16,), plo + PSZ, jnp.int32)
        lane = lax.iota(jnp.int32, 16)
        cp_lo = ps[p] // CHUNK
        cp_hi = (ps[p + 1] + CHUNK - 1) // CHUNK

        def _chunk(k, c):
            for j in range(CHUNK // 16):
                v = idx_buf[k, pl.ds(j * 16, 16)]
                ok = (v >= lo_v) & (v < hi_v)
                d = v - lo_v
                lr = jnp.where(ok, d >> 11, PROWS)
                lc = jnp.where(ok, d & (INW - 1), lane)
                plsc.addupdate_scatter(
                    pb, [lr, lc], val_buf[k, pl.ds(j * 16, 16)])
            return c

        def _sblock(bb, c):
            blk = c_lo + bb * BLK

            @pl.when(bb > 0)
            def _():
                _stage(blk)

            k_lo = jnp.maximum(cp_lo - blk, 0)
            k_hi = jnp.minimum(jnp.minimum(cp_hi, c_hi) - blk, BLK)
            lax.fori_loop(k_lo, jnp.maximum(k_lo, k_hi), _chunk, c)
            return c

        lax.fori_loop(0, nblk, _sblock, 0)
        pltpu.async_copy(pb.at[pl.ds(0, PROWS)], wout_hbm.at[pl.ds(prow, PROWS)],
                         sems[b])

    for p in (NPIECE - 2, NPIECE - 1):
        b = p % 2
        prow = wid * (NPIECE * PROWS) + p * PROWS
        pltpu.make_async_copy(
            pbufs[b].at[pl.ds(0, PROWS)], wout_hbm.at[pl.ds(prow, PROWS)],
            sems[b]).wait()


def _densify(word2, pval2, starts):
    mesh = plsc.VectorSubcoreMesh(
        core_axis_name="c", subcore_axis_name="s", num_cores=NC,
        num_subcores=NS)
    k = pl.kernel(
        _sc_densify_body,
        out_type=jax.ShapeDtypeStruct((OUT_C, INW), jnp.int32),
        mesh=mesh,
        compiler_params=pltpu.CompilerParams(needs_layout_passes=False),
        scratch_types=[
            pltpu.VMEM((BLK, CHUNK), jnp.int32),
            pltpu.VMEM((BLK, CHUNK), jnp.int32),
            pltpu.VMEM((PROWS + 1, INW), jnp.int32),
            pltpu.VMEM((PROWS + 1, INW), jnp.int32),
            pltpu.VMEM((NW, 16), jnp.int32),
            pltpu.SemaphoreType.DMA,
            pltpu.SemaphoreType.DMA,
        ],
    )
    return k(word2, pval2, starts)


def _mm_body(x_ref, w_ref, b_ref, o_ref):
    w = w_ref[...]
    we = lax.bitcast_convert_type(w << 16, jnp.float32).astype(jnp.bfloat16)
    wo = lax.bitcast_convert_type(
        w & jnp.int32(-65536), jnp.float32).astype(jnp.bfloat16)
    acc = lax.dot_general(
        x_ref[:, :INW], we, (((1,), (1,)), ((), ())),
        preferred_element_type=jnp.float32)
    acc += lax.dot_general(
        x_ref[:, INW:], wo, (((1,), (1,)), ((), ())),
        preferred_element_type=jnp.float32)
    o_ref[...] = acc + b_ref[...]


def _matmul(x, wword, bias2):
    n_blk = 16
    ob = OUT_C // n_blk
    return pl.pallas_call(
        _mm_body,
        grid=(n_blk,),
        in_specs=[
            pl.BlockSpec((256, IN_C), lambda j: (0, 0)),
            pl.BlockSpec((ob, INW), lambda j: (j, 0)),
            pl.BlockSpec((1, ob), lambda j: (0, j)),
        ],
        out_specs=pl.BlockSpec((256, ob), lambda j: (0, j)),
        out_shape=jax.ShapeDtypeStruct((256, OUT_C), jnp.float32),
    )(x.astype(jnp.bfloat16), wword, bias2)


def kernel(x, sparse_weight, bias, rows, cols):
    nnz = rows.shape[0]
    word = rows * INW + (cols & (INW - 1))
    bits = lax.bitcast_convert_type(
        sparse_weight.astype(jnp.bfloat16), jnp.uint16).astype(jnp.int32)
    pval = bits << (16 * (cols >> 11))
    blk_elems = BLK * CHUNK
    npad = blk_elems * max(1, -(-nnz // blk_elems)) + blk_elems
    word_p = jnp.pad(word, (0, npad - nnz), constant_values=TOTW)
    pval_p = jnp.pad(pval, (0, npad - nnz))
    pid = rows >> 4
    counts = jnp.sum(
        (pid[:, None] == jnp.arange(NW * NPIECE, dtype=jnp.int32)[None, :])
        .astype(jnp.float32), axis=0)
    starts = jnp.concatenate(
        [jnp.zeros((1,), jnp.int32),
         jnp.cumsum(counts).astype(jnp.int32)])
    srows = jnp.pad(
        starts[:-1].reshape(NW, NPIECE), ((0, 0), (0, 16 - NPIECE)),
        mode="constant")
    srows = srows.at[:, NPIECE].set(starts[NPIECE::NPIECE])
    wword = _densify(word_p.reshape(-1, CHUNK), pval_p.reshape(-1, CHUNK),
                     srows)
    return _matmul(x, wword, bias.reshape(1, OUT_C))


# speedup vs baseline: 1.9567x; 1.9567x over previous
import jax
import jax.numpy as jnp
from jax import lax
from jax.experimental import pallas as pl
from jax.experimental.pallas import tpu as pltpu
from jax.experimental.pallas import tpu_sc as plsc

IN_C = 4096
OUT_C = 4096
INW = IN_C // 2
TOTW = OUT_C * INW
NC = 2
NS = 16
NW = NC * NS
REG = TOTW // NW
CHUNK = 128
BLK = 64
PSZ = 32768
PROWS = PSZ // INW
NPIECE = REG // PSZ
SUB = 512


def _sc_densify_body(idx_hbm, val_hbm, starts_hbm, wout_hbm,
                     idx_buf, val_buf, pbuf0, pbuf1, starts_v, sem0, sem1):
    wid = lax.axis_index("s") * NC + lax.axis_index("c")
    sems = (sem0, sem1)
    pbufs = (pbuf0, pbuf1)

    pltpu.sync_copy(starts_hbm, starts_v)
    srow = starts_v[wid, :]
    kv = [srow[i] for i in range(NPIECE + 1)]
    ps_lo = [jnp.maximum(k - 1, 0) * SUB for k in kv]
    ps_hi = [k * SUB for k in kv]
    c_lo = (ps_lo[0] // CHUNK) // 8 * 8
    c_hi = (ps_hi[NPIECE] + CHUNK - 1) // CHUNK
    nblk = (c_hi - c_lo + BLK - 1) // BLK

    def _stage(blk):
        pltpu.sync_copy(idx_hbm.at[pl.ds(blk, BLK)], idx_buf)
        pltpu.sync_copy(val_hbm.at[pl.ds(blk, BLK)], val_buf)

    _stage(c_lo)

    for p in range(NPIECE):
        b = p % 2
        pb = pbufs[b]
        plo = wid * REG + p * PSZ
        prow = wid * (NPIECE * PROWS) + p * PROWS

        if p >= 2:
            prev = wid * (NPIECE * PROWS) + (p - 2) * PROWS
            pltpu.make_async_copy(
                pb.at[pl.ds(0, PROWS)], wout_hbm.at[pl.ds(prev, PROWS)],
                sems[b]).wait()
        if p > 0:
            @pl.when(nblk > 1)
            def _():
                _stage(c_lo)

        def _zf(i, c):
            for r in range(PROWS):
                pb[r, pl.ds(i * 16, 16)] = jnp.zeros((16,), jnp.int32)
            return c

        lax.fori_loop(0, INW // 16, _zf, 0)

        lo_v = jnp.full((16,), plo, jnp.int32)
        hi_v = jnp.full((16,), plo + PSZ, jnp.int32)
        lane = lax.iota(jnp.int32, 16)
        cp_lo = ps_lo[p] // CHUNK
        cp_hi = (ps_hi[p + 1] + CHUNK - 1) // CHUNK

        def _chunk(k, c):
            for j in range(CHUNK // 16):
                v = idx_buf[k, pl.ds(j * 16, 16)]
                ok = (v >= lo_v) & (v < hi_v)
                d = v - lo_v
                lr = jnp.where(ok, d >> 11, PROWS)
                lc = jnp.where(ok, d & (INW - 1), lane)
                plsc.addupdate_scatter(
                    pb, [lr, lc], val_buf[k, pl.ds(j * 16, 16)])
            return c

        def _sblock(bb, c):
            blk = c_lo + bb * BLK

            @pl.when(bb > 0)
            def _():
                _stage(blk)

            k_lo = jnp.maximum(cp_lo - blk, 0)
            k_hi = jnp.minimum(jnp.minimum(cp_hi, c_hi) - blk, BLK)
            lax.fori_loop(k_lo, jnp.maximum(k_lo, k_hi), _chunk, c)
            return c

        lax.fori_loop(0, nblk, _sblock, 0)
        pltpu.async_copy(pb.at[pl.ds(0, PROWS)], wout_hbm.at[pl.ds(prow, PROWS)],
                         sems[b])

    for p in (NPIECE - 2, NPIECE - 1):
        b = p % 2
        prow = wid * (NPIECE * PROWS) + p * PROWS
        pltpu.make_async_copy(
            pbufs[b].at[pl.ds(0, PROWS)], wout_hbm.at[pl.ds(prow, PROWS)],
            sems[b]).wait()


def _densify(word2, pval2, starts):
    mesh = plsc.VectorSubcoreMesh(
        core_axis_name="c", subcore_axis_name="s", num_cores=NC,
        num_subcores=NS)
    k = pl.kernel(
        _sc_densify_body,
        out_type=jax.ShapeDtypeStruct((OUT_C, INW), jnp.int32),
        mesh=mesh,
        compiler_params=pltpu.CompilerParams(needs_layout_passes=False),
        scratch_types=[
            pltpu.VMEM((BLK, CHUNK), jnp.int32),
            pltpu.VMEM((BLK, CHUNK), jnp.int32),
            pltpu.VMEM((PROWS + 1, INW), jnp.int32),
            pltpu.VMEM((PROWS + 1, INW), jnp.int32),
            pltpu.VMEM((NW, 16), jnp.int32),
            pltpu.SemaphoreType.DMA,
            pltpu.SemaphoreType.DMA,
        ],
    )
    return k(word2, pval2, starts)


def _mm_body(x_ref, w_ref, b_ref, o_ref):
    w = w_ref[...]
    we = lax.bitcast_convert_type(w << 16, jnp.float32).astype(jnp.bfloat16)
    wo = lax.bitcast_convert_type(
        w & jnp.int32(-65536), jnp.float32).astype(jnp.bfloat16)
    acc = lax.dot_general(
        x_ref[:, :INW], we, (((1,), (1,)), ((), ())),
        preferred_element_type=jnp.float32)
    acc += lax.dot_general(
        x_ref[:, INW:], wo, (((1,), (1,)), ((), ())),
        preferred_element_type=jnp.float32)
    o_ref[...] = acc + b_ref[...]


def _matmul(x, wword, bias2):
    n_blk = 16
    ob = OUT_C // n_blk
    return pl.pallas_call(
        _mm_body,
        grid=(n_blk,),
        in_specs=[
            pl.BlockSpec((256, IN_C), lambda j: (0, 0)),
            pl.BlockSpec((ob, INW), lambda j: (j, 0)),
            pl.BlockSpec((1, ob), lambda j: (0, j)),
        ],
        out_specs=pl.BlockSpec((256, ob), lambda j: (0, j)),
        out_shape=jax.ShapeDtypeStruct((256, OUT_C), jnp.float32),
    )(x.astype(jnp.bfloat16), wword, bias2)


def kernel(x, sparse_weight, bias, rows, cols):
    nnz = rows.shape[0]
    word = rows * INW + (cols & (INW - 1))
    bits = lax.bitcast_convert_type(
        sparse_weight.astype(jnp.bfloat16), jnp.uint16).astype(jnp.int32)
    pval = bits << (16 * (cols >> 11))
    blk_elems = BLK * CHUNK
    npad = blk_elems * max(1, -(-nnz // blk_elems)) + blk_elems
    word_p = jnp.pad(word, (0, npad - nnz), constant_values=TOTW)
    pval_p = jnp.pad(pval, (0, npad - nnz))
    rows_p = jnp.pad(rows, (0, npad - nnz), constant_values=OUT_C)
    rows_sub = rows_p[::SUB]
    bounds = jnp.arange(NW * NPIECE + 1, dtype=jnp.int32) * PROWS
    starts = jnp.sum(
        (rows_sub[None, :] < bounds[:, None]), axis=1).astype(jnp.int32)
    srows = jnp.pad(
        starts[:-1].reshape(NW, NPIECE), ((0, 0), (0, 16 - NPIECE)),
        mode="constant")
    srows = srows.at[:, NPIECE].set(starts[NPIECE::NPIECE])
    wword = _densify(word_p.reshape(-1, CHUNK), pval_p.reshape(-1, CHUNK),
                     srows)
    return _matmul(x, wword, bias.reshape(1, OUT_C))


# speedup vs baseline: 2.0173x; 1.0310x over previous
import jax
import jax.numpy as jnp
from jax import lax
from jax.experimental import pallas as pl
from jax.experimental.pallas import tpu as pltpu
from jax.experimental.pallas import tpu_sc as plsc

IN_C = 4096
OUT_C = 4096
INW = IN_C // 2
TOTW = OUT_C * INW
NC = 2
NS = 16
NW = NC * NS
REG = TOTW // NW
CHUNK = 128
BLK = 64
PSZ = 32768
PROWS = PSZ // INW
NPIECE = REG // PSZ
SUB = 512


def _sc_densify_body(idx_hbm, val_hbm, starts_hbm, wout_hbm,
                     idx_buf, val_buf, pbuf0, pbuf1, starts_v, sem0, sem1):
    wid = lax.axis_index("s") * NC + lax.axis_index("c")
    sems = (sem0, sem1)
    pbufs = (pbuf0, pbuf1)

    pltpu.sync_copy(starts_hbm, starts_v)
    srow = starts_v[wid, :]
    kv = [srow[i] for i in range(NPIECE + 1)]
    ps_lo = [jnp.maximum(k - 1, 0) * SUB for k in kv]
    ps_hi = [k * SUB for k in kv]
    c_lo = (ps_lo[0] // CHUNK) // 8 * 8
    c_hi = (ps_hi[NPIECE] + CHUNK - 1) // CHUNK
    nblk = (c_hi - c_lo + BLK - 1) // BLK

    def _stage(blk):
        pltpu.sync_copy(idx_hbm.at[pl.ds(blk, BLK)], idx_buf)
        pltpu.sync_copy(val_hbm.at[pl.ds(blk, BLK)], val_buf)

    _stage(c_lo)
    lane = lax.iota(jnp.int32, 16)
    zero16 = jnp.zeros((16,), jnp.int32)

    def _scan(pb, p, zero):
        plo = wid * REG + p * PSZ
        lo_v = jnp.full((16,), plo, jnp.int32)
        hi_v = jnp.full((16,), plo + PSZ, jnp.int32)
        cp_lo = ps_lo[p] // CHUNK
        cp_hi = (ps_hi[p + 1] + CHUNK - 1) // CHUNK

        def _chunk(k, c):
            for j in range(CHUNK // 16):
                v = idx_buf[k, pl.ds(j * 16, 16)]
                ok = (v >= lo_v) & (v < hi_v)
                d = v - lo_v
                lr = jnp.where(ok, d >> 11, PROWS)
                lc = jnp.where(ok, d & (INW - 1), lane)
                if zero:
                    plsc.store_scatter(pb, [lr, lc], zero16)
                else:
                    plsc.addupdate_scatter(
                        pb, [lr, lc], val_buf[k, pl.ds(j * 16, 16)])
            return c

        def _sblock(bb, c):
            blk = c_lo + bb * BLK

            @pl.when(bb > 0)
            def _():
                _stage(blk)

            k_lo = jnp.maximum(cp_lo - blk, 0)
            k_hi = jnp.minimum(jnp.minimum(cp_hi, c_hi) - blk, BLK)
            lax.fori_loop(k_lo, jnp.maximum(k_lo, k_hi), _chunk, c)
            return c

        lax.fori_loop(0, nblk, _sblock, 0)

    def _zfill(pb):
        def _zf(i, c):
            for r in range(PROWS):
                pb[r, pl.ds(i * 16, 16)] = zero16
            return c

        lax.fori_loop(0, INW // 16, _zf, 0)

    for p in range(NPIECE):
        b = p % 2
        pb = pbufs[b]
        prow = wid * (NPIECE * PROWS) + p * PROWS

        if p >= 2:
            prev = wid * (NPIECE * PROWS) + (p - 2) * PROWS
            pltpu.make_async_copy(
                pb.at[pl.ds(0, PROWS)], wout_hbm.at[pl.ds(prev, PROWS)],
                sems[b]).wait()

            @pl.when(nblk > 1)
            def _():
                _stage(c_lo)

            _scan(pb, p - 2, zero=True)
        else:
            _zfill(pb)
        if p > 0:
            @pl.when(nblk > 1)
            def _():
                _stage(c_lo)

        _scan(pb, p, zero=False)
        pltpu.async_copy(pb.at[pl.ds(0, PROWS)], wout_hbm.at[pl.ds(prow, PROWS)],
                         sems[b])

    for p in (NPIECE - 2, NPIECE - 1):
        b = p % 2
        prow = wid * (NPIECE * PROWS) + p * PROWS
        pltpu.make_async_copy(
            pbufs[b].at[pl.ds(0, PROWS)], wout_hbm.at[pl.ds(prow, PROWS)],
            sems[b]).wait()


def _densify(word2, pval2, starts):
    mesh = plsc.VectorSubcoreMesh(
        core_axis_name="c", subcore_axis_name="s", num_cores=NC,
        num_subcores=NS)
    k = pl.kernel(
        _sc_densify_body,
        out_type=jax.ShapeDtypeStruct((OUT_C, INW), jnp.int32),
        mesh=mesh,
        compiler_params=pltpu.CompilerParams(needs_layout_passes=False),
        scratch_types=[
            pltpu.VMEM((BLK, CHUNK), jnp.int32),
            pltpu.VMEM((BLK, CHUNK), jnp.int32),
            pltpu.VMEM((PROWS + 1, INW), jnp.int32),
            pltpu.VMEM((PROWS + 1, INW), jnp.int32),
            pltpu.VMEM((NW, 16), jnp.int32),
            pltpu.SemaphoreType.DMA,
            pltpu.SemaphoreType.DMA,
        ],
    )
    return k(word2, pval2, starts)


def _mm_body(x_ref, w_ref, b_ref, o_ref):
    w = w_ref[...]
    we = lax.bitcast_convert_type(w << 16, jnp.float32).astype(jnp.bfloat16)
    wo = lax.bitcast_convert_type(
        w & jnp.int32(-65536), jnp.float32).astype(jnp.bfloat16)
    acc = lax.dot_general(
        x_ref[:, :INW], we, (((1,), (1,)), ((), ())),
        preferred_element_type=jnp.float32)
    acc += lax.dot_general(
        x_ref[:, INW:], wo, (((1,), (1,)), ((), ())),
        preferred_element_type=jnp.float32)
    o_ref[...] = acc + b_ref[...]


def _matmul(x, wword, bias2):
    n_blk = 16
    ob = OUT_C // n_blk
    return pl.pallas_call(
        _mm_body,
        grid=(n_blk,),
        in_specs=[
            pl.BlockSpec((256, IN_C), lambda j: (0, 0)),
            pl.BlockSpec((ob, INW), lambda j: (j, 0)),
            pl.BlockSpec((1, ob), lambda j: (0, j)),
        ],
        out_specs=pl.BlockSpec((256, ob), lambda j: (0, j)),
        out_shape=jax.ShapeDtypeStruct((256, OUT_C), jnp.float32),
    )(x.astype(jnp.bfloat16), wword, bias2)


def kernel(x, sparse_weight, bias, rows, cols):
    nnz = rows.shape[0]
    word = rows * INW + (cols & (INW - 1))
    bits = lax.bitcast_convert_type(
        sparse_weight.astype(jnp.bfloat16), jnp.uint16).astype(jnp.int32)
    pval = bits << (16 * (cols >> 11))
    blk_elems = BLK * CHUNK
    npad = blk_elems * max(1, -(-nnz // blk_elems)) + blk_elems
    word_p = jnp.pad(word, (0, npad - nnz), constant_values=TOTW)
    pval_p = jnp.pad(pval, (0, npad - nnz))
    rows_p = jnp.pad(rows, (0, npad - nnz), constant_values=OUT_C)
    rows_sub = rows_p[::SUB]
    bounds = jnp.arange(NW * NPIECE + 1, dtype=jnp.int32) * PROWS
    starts = jnp.sum(
        (rows_sub[None, :] < bounds[:, None]), axis=1).astype(jnp.int32)
    srows = jnp.pad(
        starts[:-1].reshape(NW, NPIECE), ((0, 0), (0, 16 - NPIECE)),
        mode="constant")
    srows = srows.at[:, NPIECE].set(starts[NPIECE::NPIECE])
    wword = _densify(word_p.reshape(-1, CHUNK), pval_p.reshape(-1, CHUNK),
                     srows)
    return _matmul(x, wword, bias.reshape(1, OUT_C))


# speedup vs baseline: 2.1532x; 1.0674x over previous
import jax
import jax.numpy as jnp
from jax import lax
from jax.experimental import pallas as pl
from jax.experimental.pallas import tpu as pltpu
from jax.experimental.pallas import tpu_sc as plsc

IN_C = 4096
OUT_C = 4096
INW = IN_C // 2
TOTW = OUT_C * INW
NC = 2
NS = 16
NW = NC * NS
REG = TOTW // NW
CHUNK = 128
BLK = 64
PSZ = 32768
PROWS = PSZ // INW
NPIECE = REG // PSZ
SUB = 512


def _sc_densify_body(idx_hbm, val_hbm, starts_hbm, wout_hbm,
                     idx_buf, val_buf, pbuf0, pbuf1, starts_v, sem0, sem1):
    wid = lax.axis_index("s") * NC + lax.axis_index("c")
    sems = (sem0, sem1)
    pbufs = (pbuf0, pbuf1)

    pltpu.sync_copy(starts_hbm, starts_v)
    srow = starts_v[wid, :]
    kv = [srow[i] for i in range(NPIECE + 1)]
    ps_lo = [jnp.maximum(k - 1, 0) * SUB for k in kv]
    ps_hi = [k * SUB for k in kv]
    c_lo = (ps_lo[0] // CHUNK) // 8 * 8
    c_hi = (ps_hi[NPIECE] + CHUNK - 1) // CHUNK
    nblk = (c_hi - c_lo + BLK - 1) // BLK

    def _stage(blk):
        pltpu.sync_copy(idx_hbm.at[pl.ds(blk, BLK)], idx_buf)
        pltpu.sync_copy(val_hbm.at[pl.ds(blk, BLK)], val_buf)

    _stage(c_lo)
    lane = lax.iota(jnp.int32, 16)
    zero16 = jnp.zeros((16,), jnp.int32)

    def _scan(pb, p, zero):
        plo = wid * REG + p * PSZ
        lo_v = jnp.full((16,), plo, jnp.int32)
        hi_v = jnp.full((16,), plo + PSZ, jnp.int32)
        cp_lo = ps_lo[p] // CHUNK
        cp_hi = (ps_hi[p + 1] + CHUNK - 1) // CHUNK

        def _chunk(k, c):
            for j in range(CHUNK // 16):
                v = idx_buf[k, pl.ds(j * 16, 16)]
                ok = (v >= lo_v) & (v < hi_v)
                d = v - lo_v
                lr = jnp.where(ok, d >> 11, PROWS)
                lc = jnp.where(ok, d & (INW - 1), lane)
                if zero:
                    plsc.store_scatter(pb, [lr, lc], zero16)
                else:
                    plsc.addupdate_scatter(
                        pb, [lr, lc], val_buf[k, pl.ds(j * 16, 16)])
            return c

        def _sblock(bb, c):
            blk = c_lo + bb * BLK

            @pl.when(bb > 0)
            def _():
                _stage(blk)

            k_lo = jnp.maximum(cp_lo - blk, 0)
            k_hi = jnp.minimum(jnp.minimum(cp_hi, c_hi) - blk, BLK)
            lax.fori_loop(k_lo, jnp.maximum(k_lo, k_hi), _chunk, c)
            return c

        lax.fori_loop(0, nblk, _sblock, 0)

    def _zfill(pb):
        def _zf(i, c):
            for r in range(PROWS):
                pb[r, pl.ds(i * 16, 16)] = zero16
            return c

        lax.fori_loop(0, INW // 16, _zf, 0)

    for p in range(NPIECE):
        b = p % 2
        pb = pbufs[b]
        prow = wid * (NPIECE * PROWS) + p * PROWS

        if p >= 2:
            prev = wid * (NPIECE * PROWS) + (p - 2) * PROWS
            pltpu.make_async_copy(
                pb.at[pl.ds(0, PROWS)], wout_hbm.at[pl.ds(prev, PROWS)],
                sems[b]).wait()

            @pl.when(nblk > 1)
            def _():
                _stage(c_lo)

            _scan(pb, p - 2, zero=True)
        else:
            _zfill(pb)
        if p > 0:
            @pl.when(nblk > 1)
            def _():
                _stage(c_lo)

        _scan(pb, p, zero=False)
        pltpu.async_copy(pb.at[pl.ds(0, PROWS)], wout_hbm.at[pl.ds(prow, PROWS)],
                         sems[b])

    for p in (NPIECE - 2, NPIECE - 1):
        b = p % 2
        prow = wid * (NPIECE * PROWS) + p * PROWS
        pltpu.make_async_copy(
            pbufs[b].at[pl.ds(0, PROWS)], wout_hbm.at[pl.ds(prow, PROWS)],
            sems[b]).wait()


def _densify(word2, pval2, starts):
    mesh = plsc.VectorSubcoreMesh(
        core_axis_name="c", subcore_axis_name="s", num_cores=NC,
        num_subcores=NS)
    k = pl.kernel(
        _sc_densify_body,
        out_type=jax.ShapeDtypeStruct((OUT_C, INW), jnp.int32),
        mesh=mesh,
        compiler_params=pltpu.CompilerParams(needs_layout_passes=False),
        scratch_types=[
            pltpu.VMEM((BLK, CHUNK), jnp.int32),
            pltpu.VMEM((BLK, CHUNK), jnp.int32),
            pltpu.VMEM((PROWS + 1, INW), jnp.int32),
            pltpu.VMEM((PROWS + 1, INW), jnp.int32),
            pltpu.VMEM((NW, 16), jnp.int32),
            pltpu.SemaphoreType.DMA,
            pltpu.SemaphoreType.DMA,
        ],
    )
    return k(word2, pval2, starts)


def _mm_body(x_ref, w_ref, b_ref, o_ref):
    w = w_ref[...]
    we = lax.bitcast_convert_type(w << 16, jnp.float32).astype(jnp.bfloat16)
    wo = lax.bitcast_convert_type(
        w & jnp.int32(-65536), jnp.float32).astype(jnp.bfloat16)
    acc = lax.dot_general(
        x_ref[:, :INW], we, (((1,), (1,)), ((), ())),
        preferred_element_type=jnp.float32)
    acc += lax.dot_general(
        x_ref[:, INW:], wo, (((1,), (1,)), ((), ())),
        preferred_element_type=jnp.float32)
    o_ref[...] = acc + b_ref[...]


def _matmul(x, wword, bias2):
    n_blk = 8
    ob = OUT_C // n_blk
    return pl.pallas_call(
        _mm_body,
        grid=(n_blk,),
        in_specs=[
            pl.BlockSpec((256, IN_C), lambda j: (0, 0)),
            pl.BlockSpec((ob, INW), lambda j: (j, 0)),
            pl.BlockSpec((1, ob), lambda j: (0, j)),
        ],
        out_specs=pl.BlockSpec((256, ob), lambda j: (0, j)),
        out_shape=jax.ShapeDtypeStruct((256, OUT_C), jnp.float32),
    )(x.astype(jnp.bfloat16), wword, bias2)


def kernel(x, sparse_weight, bias, rows, cols):
    nnz = rows.shape[0]
    word = rows * INW + (cols & (INW - 1))
    bits = lax.bitcast_convert_type(
        sparse_weight.astype(jnp.bfloat16), jnp.uint16).astype(jnp.int32)
    pval = bits << (16 * (cols >> 11))
    blk_elems = BLK * CHUNK
    npad = blk_elems * max(1, -(-nnz // blk_elems)) + blk_elems
    word_p = jnp.pad(word, (0, npad - nnz), constant_values=TOTW)
    pval_p = jnp.pad(pval, (0, npad - nnz))
    rows_p = jnp.pad(rows, (0, npad - nnz), constant_values=OUT_C)
    rows_sub = rows_p[::SUB]
    bounds = jnp.arange(NW * NPIECE + 1, dtype=jnp.int32) * PROWS
    starts = jnp.sum(
        (rows_sub[None, :] < bounds[:, None]), axis=1).astype(jnp.int32)
    srows = jnp.pad(
        starts[:-1].reshape(NW, NPIECE), ((0, 0), (0, 16 - NPIECE)),
        mode="constant")
    srows = srows.at[:, NPIECE].set(starts[NPIECE::NPIECE])
    wword = _densify(word_p.reshape(-1, CHUNK), pval_p.reshape(-1, CHUNK),
                     srows)
    return _matmul(x, wword, bias.reshape(1, OUT_C))


# speedup vs baseline: 2.1725x; 1.0089x over previous
import jax
import jax.numpy as jnp
from jax import lax
from jax.experimental import pallas as pl
from jax.experimental.pallas import tpu as pltpu
from jax.experimental.pallas import tpu_sc as plsc

IN_C = 4096
OUT_C = 4096
INW = IN_C // 2
TOTW = OUT_C * INW
NC = 2
NS = 16
NW = NC * NS
REG = TOTW // NW
CHUNK = 128
BLK = 64
PSZ = 32768
PROWS = PSZ // INW
NPIECE = REG // PSZ
SUB = 512


def _sc_densify_body(idx_hbm, val_hbm, starts_hbm, wout_hbm,
                     idx_buf, val_buf, pbuf0, pbuf1, starts_v, sem0, sem1):
    wid = lax.axis_index("s") * NC + lax.axis_index("c")
    sems = (sem0, sem1)
    pbufs = (pbuf0, pbuf1)

    pltpu.sync_copy(starts_hbm, starts_v)
    srow = starts_v[wid, :]
    kv = [srow[i] for i in range(NPIECE + 1)]
    ps_lo = [jnp.maximum(k - 1, 0) * SUB for k in kv]
    ps_hi = [k * SUB for k in kv]
    c_lo = (ps_lo[0] // CHUNK) // 8 * 8
    c_hi = (ps_hi[NPIECE] + CHUNK - 1) // CHUNK
    nblk = (c_hi - c_lo + BLK - 1) // BLK

    def _stage(blk):
        pltpu.sync_copy(idx_hbm.at[pl.ds(blk, BLK)], idx_buf)
        pltpu.sync_copy(val_hbm.at[pl.ds(blk, BLK)], val_buf)

    _stage(c_lo)
    lane = lax.iota(jnp.int32, 16)
    zero16 = jnp.zeros((16,), jnp.int32)

    def _scan(pb, p, zero):
        plo = wid * REG + p * PSZ
        lo_v = jnp.full((16,), plo, jnp.int32)
        hi_v = jnp.full((16,), plo + PSZ, jnp.int32)
        cp_lo = ps_lo[p] // CHUNK
        cp_hi = (ps_hi[p + 1] + CHUNK - 1) // CHUNK

        def _chunk(k, c):
            for j in range(CHUNK // 16):
                v = idx_buf[k, pl.ds(j * 16, 16)]
                ok = (v >= lo_v) & (v < hi_v)
                d = v - lo_v
                lr = jnp.where(ok, d >> 11, PROWS)
                lc = jnp.where(ok, d & (INW - 1), lane)
                if zero:
                    plsc.store_scatter(pb, [lr, lc], zero16)
                else:
                    plsc.addupdate_scatter(
                        pb, [lr, lc], val_buf[k, pl.ds(j * 16, 16)])
            return c

        def _sblock(bb, c):
            blk = c_lo + bb * BLK

            @pl.when(bb > 0)
            def _():
                _stage(blk)

            k_lo = jnp.maximum(cp_lo - blk, 0)
            k_hi = jnp.minimum(jnp.minimum(cp_hi, c_hi) - blk, BLK)
            lax.fori_loop(k_lo, jnp.maximum(k_lo, k_hi), _chunk, c)
            return c

        lax.fori_loop(0, nblk, _sblock, 0)

    def _zfill(pb):
        def _zf(i, c):
            for r in range(PROWS):
                pb[r, pl.ds(i * 16, 16)] = zero16
            return c

        lax.fori_loop(0, INW // 16, _zf, 0)

    for p in range(NPIECE):
        b = p % 2
        pb = pbufs[b]
        prow = wid * (NPIECE * PROWS) + p * PROWS

        if p >= 2:
            prev = wid * (NPIECE * PROWS) + (p - 2) * PROWS
            pltpu.make_async_copy(
                pb.at[pl.ds(0, PROWS)], wout_hbm.at[pl.ds(prev, PROWS)],
                sems[b]).wait()

            @pl.when(nblk > 1)
            def _():
                _stage(c_lo)

            _scan(pb, p - 2, zero=True)
        else:
            _zfill(pb)
        if p > 0:
            @pl.when(nblk > 1)
            def _():
                _stage(c_lo)

        _scan(pb, p, zero=False)
        pltpu.async_copy(pb.at[pl.ds(0, PROWS)], wout_hbm.at[pl.ds(prow, PROWS)],
                         sems[b])

    for p in (NPIECE - 2, NPIECE - 1):
        b = p % 2
        prow = wid * (NPIECE * PROWS) + p * PROWS
        pltpu.make_async_copy(
            pbufs[b].at[pl.ds(0, PROWS)], wout_hbm.at[pl.ds(prow, PROWS)],
            sems[b]).wait()


def _densify(word2, pval2, starts):
    mesh = plsc.VectorSubcoreMesh(
        core_axis_name="c", subcore_axis_name="s", num_cores=NC,
        num_subcores=NS)
    k = pl.kernel(
        _sc_densify_body,
        out_type=jax.ShapeDtypeStruct((OUT_C, INW), jnp.int32),
        mesh=mesh,
        compiler_params=pltpu.CompilerParams(needs_layout_passes=False),
        scratch_types=[
            pltpu.VMEM((BLK, CHUNK), jnp.int32),
            pltpu.VMEM((BLK, CHUNK), jnp.int32),
            pltpu.VMEM((PROWS + 1, INW), jnp.int32),
            pltpu.VMEM((PROWS + 1, INW), jnp.int32),
            pltpu.VMEM((NW, 16), jnp.int32),
            pltpu.SemaphoreType.DMA,
            pltpu.SemaphoreType.DMA,
        ],
    )
    return k(word2, pval2, starts)


def _mm_body(x_ref, w_ref, b_ref, o_ref):
    w = w_ref[...]
    we = lax.bitcast_convert_type(w << 16, jnp.float32).astype(jnp.bfloat16)
    wo = lax.bitcast_convert_type(
        w & jnp.int32(-65536), jnp.float32).astype(jnp.bfloat16)
    acc = lax.dot_general(
        x_ref[:, :INW], we, (((1,), (1,)), ((), ())),
        preferred_element_type=jnp.float32)
    acc += lax.dot_general(
        x_ref[:, INW:], wo, (((1,), (1,)), ((), ())),
        preferred_element_type=jnp.float32)
    o_ref[...] = acc + b_ref[...]


def _matmul(x, wword, bias2):
    n_blk = 4
    ob = OUT_C // n_blk
    return pl.pallas_call(
        _mm_body,
        grid=(n_blk,),
        in_specs=[
            pl.BlockSpec((256, IN_C), lambda j: (0, 0)),
            pl.BlockSpec((ob, INW), lambda j: (j, 0)),
            pl.BlockSpec((1, ob), lambda j: (0, j)),
        ],
        out_specs=pl.BlockSpec((256, ob), lambda j: (0, j)),
        out_shape=jax.ShapeDtypeStruct((256, OUT_C), jnp.float32),
    )(x.astype(jnp.bfloat16), wword, bias2)


def kernel(x, sparse_weight, bias, rows, cols):
    nnz = rows.shape[0]
    word = rows * INW + (cols & (INW - 1))
    bits = lax.bitcast_convert_type(
        sparse_weight.astype(jnp.bfloat16), jnp.uint16).astype(jnp.int32)
    pval = bits << (16 * (cols >> 11))
    blk_elems = BLK * CHUNK
    npad = blk_elems * max(1, -(-nnz // blk_elems)) + blk_elems
    word_p = jnp.pad(word, (0, npad - nnz), constant_values=TOTW)
    pval_p = jnp.pad(pval, (0, npad - nnz))
    rows_p = jnp.pad(rows, (0, npad - nnz), constant_values=OUT_C)
    rows_sub = rows_p[::SUB]
    bounds = jnp.arange(NW * NPIECE + 1, dtype=jnp.int32) * PROWS
    starts = jnp.sum(
        (rows_sub[None, :] < bounds[:, None]), axis=1).astype(jnp.int32)
    srows = jnp.pad(
        starts[:-1].reshape(NW, NPIECE), ((0, 0), (0, 16 - NPIECE)),
        mode="constant")
    srows = srows.at[:, NPIECE].set(starts[NPIECE::NPIECE])
    wword = _densify(word_p.reshape(-1, CHUNK), pval_p.reshape(-1, CHUNK),
                     srows)
    return _matmul(x, wword, bias.reshape(1, OUT_C))
